# baseline (speedup 1.0000x reference)
import functools

import jax
import jax.numpy as jnp
import numpy as np
from jax import lax
from jax.experimental import pallas as pl
from jax.experimental.pallas import tpu as pltpu
from jax.experimental.pallas import tpu_sc as plsc

_N = 10000
_E = 320000
_D = 128
_HALF = 64
_NT = 16
_CH = 128
_NCHUNK = 160
_NBUF = 4
_EPT = _NCHUNK * _CH
_NPAD = 10240
_ROWS_PER_TILE = _NPAD // _NT
_ACC_ROWS = _NPAD

def _make_spmm_body(nrounds, in_slot):
    def body(h_in_hbm, src_hbm, dst_hbm, zeros_hbm, out_hbm,
             src_v, dst_v, b0, b1, b2, b3, acc, g0, g1, g2, g3):
        c = lax.axis_index("c")
        s = lax.axis_index("s")
        row0 = s * _ROWS_PER_TILE
        rows = pl.ds(row0, _ROWS_PER_TILE)
        bufs = (b0, b1, b2, b3)
        gsem = (g0, g1, g2, g3)
        pltpu.sync_copy(src_hbm.at[s], src_v)
        pltpu.sync_copy(dst_hbm.at[s], dst_v)
        pltpu.sync_copy(zeros_hbm, acc.at[rows])
        plsc.subcore_barrier()

        for r in range(nrounds):
            if r == 0:
                src_hbm_r = (h_in_hbm.at[c] if in_slot is None
                             else h_in_hbm.at[in_slot, c])
            else:
                src_hbm_r = out_hbm.at[r - 1, c]

            def wait_gather(b, src_hbm_r=src_hbm_r):
                pltpu.make_async_copy(
                    src_hbm_r.at[pl.ds(0, _CH)], bufs[b], gsem[b]).wait()

            for b in range(_NBUF):
                pltpu.async_copy(
                    src_hbm_r.at[src_v.at[b]], bufs[b], gsem[b])

            def loop(j, carry, src_hbm_r=src_hbm_r,
                     wait_gather=wait_gather):
                for b in range(_NBUF):
                    t = _NBUF * j + b
                    wait_gather(b)
                    pltpu.sync_copy(bufs[b], acc.at[dst_v.at[t]],
                                    add=True)
                    pltpu.async_copy(
                        src_hbm_r.at[src_v.at[t + _NBUF]], bufs[b],
                        gsem[b])
                return carry

            lax.fori_loop(0, _NCHUNK // _NBUF, loop, jnp.int32(0))
            for b in range(_NBUF):
                wait_gather(b)
            plsc.subcore_barrier()
            pltpu.sync_copy(acc.at[rows], out_hbm.at[r, c, rows])
            if r < nrounds - 1:
                pltpu.sync_copy(zeros_hbm, acc.at[rows])
                plsc.subcore_barrier()

    return body


@functools.cache
def _spmm_kernel(nrounds, in_slot):
    mesh = plsc.VectorSubcoreMesh(
        core_axis_name="c", subcore_axis_name="s",
        num_cores=2, num_subcores=_NT)
    return pl.kernel(
        _make_spmm_body(nrounds, in_slot),
        out_type=jax.ShapeDtypeStruct((nrounds, 2, _NPAD, _HALF),
                                      jnp.float32),
        mesh=mesh,
        scratch_types=[
            pltpu.VMEM((_NCHUNK + _NBUF, _CH), jnp.int32),
            pltpu.VMEM((_NCHUNK, _CH), jnp.int32),
        ] + [pltpu.VMEM((_CH, _HALF), jnp.float32)] * _NBUF
        + [pltpu.VMEM_SHARED((_ACC_ROWS, _HALF), jnp.float32)]
        + [pltpu.SemaphoreType.DMA] * _NBUF,
        compiler_params=pltpu.CompilerParams(
            use_tc_tiling_on_sc=False,
            disable_bounds_checks=True,
        ),
    )


_BLK = 1000


def _dense1_body(x_ref, hs_ref, wa_ref, wb_ref, out_ref):
    x = x_ref[...]
    acc_a = jnp.dot(x, wa_ref[0] + wa_ref[1] + wa_ref[2],
                    preferred_element_type=jnp.float32)
    acc_b = jnp.dot(x, wb_ref[0] + wb_ref[1] + wb_ref[2],
                    preferred_element_type=jnp.float32)
    for r in range(2):
        for c in range(2):
            h = hs_ref[r, c]
            acc_a += jnp.dot(h, wa_ref[3 + r, c * _HALF:(c + 1) * _HALF, :],
                             preferred_element_type=jnp.float32)
            acc_b += jnp.dot(h, wb_ref[3 + r, c * _HALF:(c + 1) * _HALF, :],
                             preferred_element_type=jnp.float32)
    out_ref[:, :_D] = acc_a
    out_ref[:, _D:] = acc_b


_dense1_call = pl.pallas_call(
    _dense1_body,
    grid=(_N // _BLK,),
    in_specs=[
        pl.BlockSpec((_BLK, _D), lambda i: (i, 0)),
        pl.BlockSpec((3, 2, _BLK, _HALF), lambda i: (0, 0, i, 0)),
        pl.BlockSpec((6, _D, _D), lambda i: (0, 0, 0)),
        pl.BlockSpec((6, _D, _D), lambda i: (0, 0, 0)),
    ],
    out_specs=pl.BlockSpec((_BLK, 2 * _D), lambda i: (i, 0)),
    out_shape=jax.ShapeDtypeStruct((_N, 2 * _D), jnp.float32),
)


def _dense2_body(part_ref, h4_ref, wa_ref, wb_ref,
                 ag_ref, ab_ref, am_ref, av_ref,
                 bg_ref, bb_ref, bm_ref, bv_ref, out_ref):
    acc_a = part_ref[:, :_D]
    acc_b = part_ref[:, _D:]
    for c in range(2):
        h = h4_ref[0, c]
        acc_a += jnp.dot(h, wa_ref[5, c * _HALF:(c + 1) * _HALF, :],
                         preferred_element_type=jnp.float32)
        acc_b += jnp.dot(h, wb_ref[5, c * _HALF:(c + 1) * _HALF, :],
                         preferred_element_type=jnp.float32)
    alpha = jnp.maximum(acc_a, 0.0)
    alpha = ((alpha - am_ref[...]) / jnp.sqrt(av_ref[...] + 1e-3)
             * ag_ref[...] + ab_ref[...])
    beta = ((acc_b - bm_ref[...]) / jnp.sqrt(bv_ref[...] + 1e-3)
            * bg_ref[...] + bb_ref[...])
    out_ref[:, :_D] = alpha
    out_ref[:, _D:] = beta


_dense2_call = pl.pallas_call(
    _dense2_body,
    grid=(_N // _BLK,),
    in_specs=[
        pl.BlockSpec((_BLK, 2 * _D), lambda i: (i, 0)),
        pl.BlockSpec((1, 2, _BLK, _HALF), lambda i: (0, 0, i, 0)),
        pl.BlockSpec((6, _D, _D), lambda i: (0, 0, 0)),
        pl.BlockSpec((6, _D, _D), lambda i: (0, 0, 0)),
    ] + [pl.BlockSpec((1, _D), lambda i: (0, 0))] * 8,
    out_specs=pl.BlockSpec((_BLK, 2 * _D), lambda i: (i, 0)),
    out_shape=jax.ShapeDtypeStruct((_N, 2 * _D), jnp.float32),
)


def kernel(x, edge_index, Wa, Wb,
           bn_a_gamma, bn_a_beta, bn_a_mean, bn_a_var,
           bn_b_gamma, bn_b_beta, bn_b_mean, bn_b_var):
    dst = edge_index[0]
    src = edge_index[1]
    pad = _NT * _EPT - _E
    pad_src = np.arange(pad, dtype=np.int32) % _N
    pad_dst = _N + np.arange(pad, dtype=np.int32) % 16
    src_p = jnp.concatenate([src, jnp.asarray(pad_src)]).reshape(
        _NT, _NCHUNK, _CH)
    dst_p = jnp.concatenate([dst, jnp.asarray(pad_dst)]).reshape(
        _NT, _NCHUNK, _CH)
    dummy = jnp.asarray(
        (np.arange(_NT * _NBUF * _CH, dtype=np.int32) % _N
         ).reshape(_NT, _NBUF, _CH))
    src_p = jnp.concatenate([src_p, dummy], axis=1)
    xcols = x.reshape(_N, 2, _HALF).transpose(1, 0, 2)
    zeros = jnp.zeros((_ROWS_PER_TILE, _HALF), jnp.float32)

    hA = _spmm_kernel(3, None)(xcols, src_p, dst_p, zeros)
    h4 = _spmm_kernel(1, 2)(hA, src_p, dst_p, zeros)
    part = _dense1_call(x, hA, Wa, Wb)

    bn2 = [b.reshape(1, _D) for b in (
        bn_a_gamma, bn_a_beta, bn_a_mean, bn_a_var,
        bn_b_gamma, bn_b_beta, bn_b_mean, bn_b_var)]
    return _dense2_call(part, h4, Wa, Wb, *bn2)


# speedup vs baseline: 1.0154x; 1.0154x over previous
import functools

import jax
import jax.numpy as jnp
import numpy as np
from jax import lax
from jax.experimental import pallas as pl
from jax.experimental.pallas import tpu as pltpu
from jax.experimental.pallas import tpu_sc as plsc

_N = 10000
_E = 320000
_D = 128
_HALF = 64
_NT = 16
_CH = 128
_NCHUNK = 160
_NBUF = 4
_EPT = _NCHUNK * _CH
_NPAD = 10240
_ROWS_PER_TILE = _NPAD // _NT
_ACC_ROWS = _NPAD

def _make_spmm_body(nrounds, in_slot):
    def body(h_in_hbm, idx_hbm, dummy_hbm, zeros_hbm, out_hbm,
             src_v, dst_v, b0, b1, b2, b3, acc, g0, g1, g2, g3):
        c = lax.axis_index("c")
        s = lax.axis_index("s")
        row0 = s * _ROWS_PER_TILE
        rows = pl.ds(row0, _ROWS_PER_TILE)
        bufs = (b0, b1, b2, b3)
        gsem = (g0, g1, g2, g3)
        pltpu.sync_copy(idx_hbm.at[0, s], dst_v)
        pltpu.sync_copy(idx_hbm.at[1, s], src_v.at[pl.ds(0, _NCHUNK)])
        pltpu.sync_copy(dummy_hbm.at[s], src_v.at[pl.ds(_NCHUNK, _NBUF)])
        pltpu.sync_copy(zeros_hbm, acc.at[rows])
        plsc.subcore_barrier()

        for r in range(nrounds):
            if r == 0:
                src_hbm_r = (h_in_hbm.at[c] if in_slot is None
                             else h_in_hbm.at[in_slot, c])
            else:
                src_hbm_r = out_hbm.at[r - 1, c]

            def wait_gather(b, src_hbm_r=src_hbm_r):
                pltpu.make_async_copy(
                    src_hbm_r.at[pl.ds(0, _CH)], bufs[b], gsem[b]).wait()

            for b in range(_NBUF):
                pltpu.async_copy(
                    src_hbm_r.at[src_v.at[b]], bufs[b], gsem[b])

            def loop(j, carry, src_hbm_r=src_hbm_r,
                     wait_gather=wait_gather):
                for b in range(_NBUF):
                    t = _NBUF * j + b
                    wait_gather(b)
                    pltpu.sync_copy(bufs[b], acc.at[dst_v.at[t]],
                                    add=True)
                    pltpu.async_copy(
                        src_hbm_r.at[src_v.at[t + _NBUF]], bufs[b],
                        gsem[b])
                return carry

            lax.fori_loop(0, _NCHUNK // _NBUF, loop, jnp.int32(0))
            for b in range(_NBUF):
                wait_gather(b)
            plsc.subcore_barrier()
            pltpu.sync_copy(acc.at[rows], out_hbm.at[r, c, rows])
            if r < nrounds - 1:
                pltpu.sync_copy(zeros_hbm, acc.at[rows])
                plsc.subcore_barrier()

    return body


@functools.cache
def _spmm_kernel(nrounds, in_slot):
    mesh = plsc.VectorSubcoreMesh(
        core_axis_name="c", subcore_axis_name="s",
        num_cores=2, num_subcores=_NT)
    return pl.kernel(
        _make_spmm_body(nrounds, in_slot),
        out_type=jax.ShapeDtypeStruct((nrounds, 2, _NPAD, _HALF),
                                      jnp.float32),
        mesh=mesh,
        scratch_types=[
            pltpu.VMEM((_NCHUNK + _NBUF, _CH), jnp.int32),
            pltpu.VMEM((_NCHUNK, _CH), jnp.int32),
        ] + [pltpu.VMEM((_CH, _HALF), jnp.float32)] * _NBUF
        + [pltpu.VMEM_SHARED((_ACC_ROWS, _HALF), jnp.float32)]
        + [pltpu.SemaphoreType.DMA] * _NBUF,
        compiler_params=pltpu.CompilerParams(
            use_tc_tiling_on_sc=False,
            disable_bounds_checks=True,
        ),
    )


_BLK = 1000


def _dense1_body(x_ref, hs_ref, wa_ref, wb_ref, out_ref):
    x = x_ref[...]
    acc_a = jnp.dot(x, wa_ref[0] + wa_ref[1] + wa_ref[2],
                    preferred_element_type=jnp.float32)
    acc_b = jnp.dot(x, wb_ref[0] + wb_ref[1] + wb_ref[2],
                    preferred_element_type=jnp.float32)
    for r in range(2):
        for c in range(2):
            h = hs_ref[r, c]
            acc_a += jnp.dot(h, wa_ref[3 + r, c * _HALF:(c + 1) * _HALF, :],
                             preferred_element_type=jnp.float32)
            acc_b += jnp.dot(h, wb_ref[3 + r, c * _HALF:(c + 1) * _HALF, :],
                             preferred_element_type=jnp.float32)
    out_ref[:, :_D] = acc_a
    out_ref[:, _D:] = acc_b


_dense1_call = pl.pallas_call(
    _dense1_body,
    grid=(_N // _BLK,),
    in_specs=[
        pl.BlockSpec((_BLK, _D), lambda i: (i, 0)),
        pl.BlockSpec((3, 2, _BLK, _HALF), lambda i: (0, 0, i, 0)),
        pl.BlockSpec((6, _D, _D), lambda i: (0, 0, 0)),
        pl.BlockSpec((6, _D, _D), lambda i: (0, 0, 0)),
    ],
    out_specs=pl.BlockSpec((_BLK, 2 * _D), lambda i: (i, 0)),
    out_shape=jax.ShapeDtypeStruct((_N, 2 * _D), jnp.float32),
)


def _dense2_body(part_ref, h4_ref, wa_ref, wb_ref,
                 ag_ref, ab_ref, am_ref, av_ref,
                 bg_ref, bb_ref, bm_ref, bv_ref, out_ref):
    acc_a = part_ref[:, :_D]
    acc_b = part_ref[:, _D:]
    for c in range(2):
        h = h4_ref[0, c]
        acc_a += jnp.dot(h, wa_ref[5, c * _HALF:(c + 1) * _HALF, :],
                         preferred_element_type=jnp.float32)
        acc_b += jnp.dot(h, wb_ref[5, c * _HALF:(c + 1) * _HALF, :],
                         preferred_element_type=jnp.float32)
    alpha = jnp.maximum(acc_a, 0.0)
    alpha = ((alpha - am_ref[...]) / jnp.sqrt(av_ref[...] + 1e-3)
             * ag_ref[...] + ab_ref[...])
    beta = ((acc_b - bm_ref[...]) / jnp.sqrt(bv_ref[...] + 1e-3)
            * bg_ref[...] + bb_ref[...])
    out_ref[:, :_D] = alpha
    out_ref[:, _D:] = beta


_dense2_call = pl.pallas_call(
    _dense2_body,
    grid=(_N // _BLK,),
    in_specs=[
        pl.BlockSpec((_BLK, 2 * _D), lambda i: (i, 0)),
        pl.BlockSpec((1, 2, _BLK, _HALF), lambda i: (0, 0, i, 0)),
        pl.BlockSpec((6, _D, _D), lambda i: (0, 0, 0)),
        pl.BlockSpec((6, _D, _D), lambda i: (0, 0, 0)),
    ] + [pl.BlockSpec((1, _D), lambda i: (0, 0))] * 8,
    out_specs=pl.BlockSpec((_BLK, 2 * _D), lambda i: (i, 0)),
    out_shape=jax.ShapeDtypeStruct((_N, 2 * _D), jnp.float32),
)


def kernel(x, edge_index, Wa, Wb,
           bn_a_gamma, bn_a_beta, bn_a_mean, bn_a_var,
           bn_b_gamma, bn_b_beta, bn_b_mean, bn_b_var):
    pad = _NT * _EPT - _E
    pads = np.stack([_N + np.arange(pad, dtype=np.int32) % 16,
                     np.arange(pad, dtype=np.int32) % _N])
    ei_p = jnp.concatenate([edge_index, jnp.asarray(pads)], axis=1
                           ).reshape(2, _NT, _NCHUNK, _CH)
    dummy = jnp.asarray(
        (np.arange(_NT * _NBUF * _CH, dtype=np.int32) % _N
         ).reshape(_NT, _NBUF, _CH))
    xcols = x.reshape(_N, 2, _HALF).transpose(1, 0, 2)
    zeros = jnp.zeros((_ROWS_PER_TILE, _HALF), jnp.float32)

    hA = _spmm_kernel(3, None)(xcols, ei_p, dummy, zeros)
    h4 = _spmm_kernel(1, 2)(hA, ei_p, dummy, zeros)
    part = _dense1_call(x, hA, Wa, Wb)

    bn2 = [b.reshape(1, _D) for b in (
        bn_a_gamma, bn_a_beta, bn_a_mean, bn_a_var,
        bn_b_gamma, bn_b_beta, bn_b_mean, bn_b_var)]
    return _dense2_call(part, h4, Wa, Wb, *bn2)
